# baseline (speedup 1.0000x reference)
import functools

import jax
import jax.numpy as jnp
from jax import lax
from jax.experimental import pallas as pl
from jax.experimental.pallas import tpu as pltpu
from jax.experimental.pallas import tpu_sc as plsc

EMBEDDING_DIM = 64
CHUNK = 128


@functools.lru_cache(maxsize=None)
def _build_gather(n_total: int, dim: int, chunk: int):
    mesh = plsc.VectorSubcoreMesh(core_axis_name="c", subcore_axis_name="s")
    n_workers = mesh.num_cores * mesh.num_subcores
    assert n_total % n_workers == 0
    rows_per_w = n_total // n_workers
    assert rows_per_w % chunk == 0
    n_chunks = rows_per_w // chunk

    @functools.partial(
        pl.kernel,
        out_type=jax.ShapeDtypeStruct((n_total, dim), jnp.float32),
        mesh=mesh,
        scratch_types=[
            pltpu.VMEM((chunk,), jnp.int32),
            pltpu.VMEM((chunk, dim), jnp.float32),
            pltpu.SemaphoreType.DMA,
        ],
        compiler_params=pltpu.CompilerParams(use_tc_tiling_on_sc=False),
    )
    def gather_kernel(idx_hbm, table_hbm, out_hbm, idx_v, rows_v, sem):
        wid = lax.axis_index("s") * mesh.num_cores + lax.axis_index("c")
        base = wid * rows_per_w

        def body(g, carry):
            off = base + g * chunk
            pltpu.sync_copy(idx_hbm.at[pl.ds(off, chunk)], idx_v)
            pltpu.async_copy(table_hbm.at[idx_v], rows_v, sem).wait()
            pltpu.sync_copy(rows_v, out_hbm.at[pl.ds(off, chunk)])
            return carry

        lax.fori_loop(0, n_chunks, body, 0)

    return gather_kernel


def kernel(input, weight):
    b, f = input.shape
    idx = input.reshape(b * f).astype(jnp.int32)
    out = _build_gather(b * f, weight.shape[1], CHUNK)(idx, weight)
    return out.reshape(b, f, weight.shape[1])


# speedup vs baseline: 1.1229x; 1.1229x over previous
import functools

import jax
import jax.numpy as jnp
from jax import lax
from jax.experimental import pallas as pl
from jax.experimental.pallas import tpu as pltpu
from jax.experimental.pallas import tpu_sc as plsc

EMBEDDING_DIM = 64
CHUNK = 512
NBUF = 2


@functools.lru_cache(maxsize=None)
def _build_gather(n_total: int, dim: int, chunk: int, nbuf: int):
    mesh = plsc.VectorSubcoreMesh(core_axis_name="c", subcore_axis_name="s")
    n_workers = mesh.num_cores * mesh.num_subcores
    assert n_total % n_workers == 0
    rows_per_w = n_total // n_workers
    assert rows_per_w % chunk == 0
    n_chunks = rows_per_w // chunk
    assert n_chunks % nbuf == 0

    @functools.partial(
        pl.kernel,
        out_type=jax.ShapeDtypeStruct((n_total, dim), jnp.float32),
        mesh=mesh,
        scratch_types=[
            [pltpu.VMEM((chunk,), jnp.int32) for _ in range(nbuf)],
            [pltpu.VMEM((chunk, dim), jnp.float32) for _ in range(nbuf)],
            [pltpu.SemaphoreType.DMA for _ in range(nbuf)],
        ],
        compiler_params=pltpu.CompilerParams(use_tc_tiling_on_sc=False),
    )
    def gather_kernel(idx_hbm, table_hbm, out_hbm, idx_v, rows_v, gsem):
        wid = lax.axis_index("s") * mesh.num_cores + lax.axis_index("c")
        base = wid * rows_per_w

        for b in range(nbuf):
            off = base + b * chunk
            pltpu.sync_copy(idx_hbm.at[pl.ds(off, chunk)], idx_v[b])
            pltpu.async_copy(table_hbm.at[idx_v[b]], rows_v[b], gsem[b])

        def body(grp, carry):
            g0 = grp * nbuf
            for b in range(nbuf):
                g = g0 + b
                pltpu.make_async_copy(table_hbm.at[idx_v[b]], rows_v[b],
                                      gsem[b]).wait()
                pltpu.sync_copy(rows_v[b],
                                out_hbm.at[pl.ds(base + g * chunk, chunk)])

                @pl.when(g + nbuf < n_chunks)
                def _():
                    off = base + (g + nbuf) * chunk
                    pltpu.sync_copy(idx_hbm.at[pl.ds(off, chunk)], idx_v[b])
                    pltpu.async_copy(table_hbm.at[idx_v[b]], rows_v[b],
                                     gsem[b])

            return carry

        lax.fori_loop(0, n_chunks // nbuf, body, 0)

    return gather_kernel


def kernel(input, weight):
    b, f = input.shape
    idx = input.reshape(b * f).astype(jnp.int32)
    out = _build_gather(b * f, weight.shape[1], CHUNK, NBUF)(idx, weight)
    return out.reshape(b, f, weight.shape[1])


# speedup vs baseline: 1.1286x; 1.0051x over previous
import functools

import jax
import jax.numpy as jnp
from jax import lax
from jax.experimental import pallas as pl
from jax.experimental.pallas import tpu as pltpu
from jax.experimental.pallas import tpu_sc as plsc

EMBEDDING_DIM = 64
CHUNK = 416
NBUF = 4


@functools.lru_cache(maxsize=None)
def _build_gather(n_total: int, dim: int, chunk: int, nbuf: int):
    mesh = plsc.VectorSubcoreMesh(core_axis_name="c", subcore_axis_name="s")
    n_workers = mesh.num_cores * mesh.num_subcores
    assert n_total % n_workers == 0
    rows_per_w = n_total // n_workers
    assert rows_per_w % chunk == 0
    n_chunks = rows_per_w // chunk
    assert n_chunks % nbuf == 0

    @functools.partial(
        pl.kernel,
        out_type=jax.ShapeDtypeStruct((n_total, dim), jnp.float32),
        mesh=mesh,
        scratch_types=[
            [pltpu.VMEM((chunk,), jnp.int32) for _ in range(nbuf)],
            [pltpu.VMEM((chunk, dim), jnp.float32) for _ in range(nbuf)],
            [pltpu.SemaphoreType.DMA for _ in range(nbuf)],
            [pltpu.SemaphoreType.DMA for _ in range(nbuf)],
        ],
        compiler_params=pltpu.CompilerParams(use_tc_tiling_on_sc=False),
    )
    def gather_kernel(idx_hbm, table_hbm, out_hbm, idx_v, rows_v, gsem, ssem):
        wid = lax.axis_index("s") * mesh.num_cores + lax.axis_index("c")
        base = wid * rows_per_w

        for b in range(nbuf):
            off = base + b * chunk
            pltpu.sync_copy(idx_hbm.at[pl.ds(off, chunk)], idx_v[b])
            pltpu.async_copy(table_hbm.at[idx_v[b]], rows_v[b], gsem[b])

        def body(grp, carry):
            g0 = grp * nbuf
            for b in range(nbuf):
                g = g0 + b
                out_slc = out_hbm.at[pl.ds(base + g * chunk, chunk)]
                pltpu.make_async_copy(table_hbm.at[idx_v[b]], rows_v[b],
                                      gsem[b]).wait()
                pltpu.async_copy(rows_v[b], out_slc, ssem[b])

                @pl.when(g + nbuf < n_chunks)
                def _():
                    off = base + (g + nbuf) * chunk
                    pltpu.sync_copy(idx_hbm.at[pl.ds(off, chunk)], idx_v[b])
                    pltpu.make_async_copy(rows_v[b], out_slc, ssem[b]).wait()
                    pltpu.async_copy(table_hbm.at[idx_v[b]], rows_v[b],
                                     gsem[b])

            return carry

        lax.fori_loop(0, n_chunks // nbuf, body, 0)

        for b in range(nbuf):
            g = n_chunks - nbuf + b
            out_slc = out_hbm.at[pl.ds(base + g * chunk, chunk)]
            pltpu.make_async_copy(rows_v[b], out_slc, ssem[b]).wait()

    return gather_kernel


def kernel(input, weight):
    b, f = input.shape
    idx = input.reshape(b * f).astype(jnp.int32)
    out = _build_gather(b * f, weight.shape[1], CHUNK, NBUF)(idx, weight)
    return out.reshape(b, f, weight.shape[1])
